# baseline (speedup 1.0000x reference)
import functools

import jax
import jax.numpy as jnp
from jax.experimental import pallas as pl
from jax.experimental.pallas import tpu as pltpu

INPUT_DIM = 256
EMB_DIM = 128
HIDDEN_DIM = 512
BATCH = 128
SEQ = 200
H4 = 4 * HIDDEN_DIM
CHUNK = 8


def _lstm_kernel(urls_ref, urlsr_ref, emb_ref, wihf_ref, whhf_ref, bf_ref,
                 wihb_ref, whhb_ref, bb_ref,
                 fcw_ref, fcb_ref, fc1w_ref, fc1b_ref,
                 fc2w_ref, fc2b_ref, fc3w_ref, fc3b_ref,
                 out_ref, aux_ref,
                 mf_scr, mb_scr, gif_scr, gib_scr):
    f32 = jnp.float32
    bf16 = jnp.bfloat16
    rows = CHUNK * BATCH

    emb = emb_ref[...]
    mf_scr[...] = (jnp.dot(emb, wihf_ref[...], preferred_element_type=f32)
                   + bf_ref[...]).astype(bf16)
    mb_scr[...] = (jnp.dot(emb, wihb_ref[...], preferred_element_type=f32)
                   + bb_ref[...]).astype(bf16)

    iota = jax.lax.broadcasted_iota(jnp.int32, (rows // 2, INPUT_DIM), 1)

    whhf = whhf_ref[...]
    whhb = whhb_ref[...]
    mf = mf_scr[...]
    mb = mb_scr[...]

    def sig(x):
        return 0.5 * jnp.tanh(0.5 * x) + 0.5

    def step_dir(gin, h, c, whh):
        gates = jnp.dot(h, whh, preferred_element_type=f32) + gin.astype(f32)
        i = sig(gates[:, 0:HIDDEN_DIM])
        f = sig(gates[:, HIDDEN_DIM:2 * HIDDEN_DIM])
        g = jnp.tanh(gates[:, 2 * HIDDEN_DIM:3 * HIDDEN_DIM])
        o = sig(gates[:, 3 * HIDDEN_DIM:])
        c = f * c + i * g
        h = (o * jnp.tanh(c)).astype(bf16)
        return h, c

    half = rows // 2

    def chunk_body(k, carry):
        def gin_half(src_ref, m_tab, dst_ref, base, off):
            ids = src_ref[pl.ds(base + off, half), :]
            dst_ref[off:off + half, :] = jnp.dot(
                (ids == iota).astype(bf16), m_tab,
                preferred_element_type=f32).astype(bf16)

        hf, cf, hb, cb = carry
        gin_half(urls_ref, mf, gif_scr, k * rows, 0)
        gin_half(urlsr_ref, mb, gib_scr, k * rows, 0)
        for j in range(CHUNK):
            if j == CHUNK // 2:
                gin_half(urls_ref, mf, gif_scr, k * rows, half)
                gin_half(urlsr_ref, mb, gib_scr, k * rows, half)
            gf = gif_scr[j * BATCH:(j + 1) * BATCH, :]
            gb = gib_scr[j * BATCH:(j + 1) * BATCH, :]
            hf, cf = step_dir(gf, hf, cf, whhf)
            hb, cb = step_dir(gb, hb, cb, whhb)
        return hf, cf, hb, cb

    hf = jnp.zeros((BATCH, HIDDEN_DIM), bf16)
    hb = jnp.zeros((BATCH, HIDDEN_DIM), bf16)
    cf = jnp.zeros((BATCH, HIDDEN_DIM), f32)
    cb = jnp.zeros((BATCH, HIDDEN_DIM), f32)
    hf, cf, hb, cb = jax.lax.fori_loop(0, SEQ // CHUNK, chunk_body,
                                       (hf, cf, hb, cb))

    hidden = jnp.concatenate([hf, hb], axis=1)
    aux_ref[...] = jnp.dot(hidden, fcw_ref[...], preferred_element_type=f32) + fcb_ref[...]
    o1 = (jnp.dot(hidden, fc1w_ref[...], preferred_element_type=f32)
          + fc1b_ref[...]).astype(bf16)
    o2 = (jnp.dot(o1, fc2w_ref[...], preferred_element_type=f32)
          + fc2b_ref[...]).astype(bf16)
    out_ref[...] = jnp.dot(o2, fc3w_ref[...], preferred_element_type=f32) + fc3b_ref[...]


@functools.partial(jax.jit, static_argnames=("interpret",))
def _run(urls, emb_table, W_ih_f, W_hh_f, b_f, W_ih_b, W_hh_b, b_b,
         fc_w, fc_b, fc1_w, fc1_b, fc2_w, fc2_b, fc3_w, fc3_b,
         interpret=False):
    urls_flat = urls.T.reshape(SEQ * BATCH, 1).astype(jnp.int32)
    urls_rev = urls.T[::-1].reshape(SEQ * BATCH, 1).astype(jnp.int32)
    f32 = jnp.float32
    bf16 = jnp.bfloat16
    args = (
        urls_flat,
        urls_rev,
        emb_table,
        W_ih_f.T, W_hh_f.T.astype(bf16), b_f.reshape(1, H4),
        W_ih_b.T, W_hh_b.T.astype(bf16), b_b.reshape(1, H4),
        fc_w.T.astype(bf16), fc_b.reshape(1, 1),
        fc1_w.T.astype(bf16), fc1_b.reshape(1, H4),
        fc2_w.T.astype(bf16), fc2_b.reshape(1, 2 * HIDDEN_DIM),
        fc3_w.T.astype(bf16), fc3_b.reshape(1, 2),
    )
    out, aux = pl.pallas_call(
        _lstm_kernel,
        out_shape=(
            jax.ShapeDtypeStruct((BATCH, 2), f32),
            jax.ShapeDtypeStruct((BATCH, 1), f32),
        ),
        scratch_shapes=[
            pltpu.VMEM((INPUT_DIM, H4), bf16),
            pltpu.VMEM((INPUT_DIM, H4), bf16),
            pltpu.VMEM((CHUNK * BATCH, H4), bf16),
            pltpu.VMEM((CHUNK * BATCH, H4), bf16),
        ],
        interpret=interpret,
    )(*args)
    return out, aux[:, 0]


def kernel(urls, emb_table, W_ih_f, W_hh_f, b_f, W_ih_b, W_hh_b, b_b,
           fc_w, fc_b, fc1_w, fc1_b, fc2_w, fc2_b, fc3_w, fc3_b):
    return _run(urls, emb_table, W_ih_f, W_hh_f, b_f, W_ih_b, W_hh_b, b_b,
                fc_w, fc_b, fc1_w, fc1_b, fc2_w, fc2_b, fc3_w, fc3_b)


# speedup vs baseline: 1.0027x; 1.0027x over previous
import functools

import jax
import jax.numpy as jnp
from jax.experimental import pallas as pl
from jax.experimental.pallas import tpu as pltpu

INPUT_DIM = 256
EMB_DIM = 128
HIDDEN_DIM = 512
BATCH = 128
SEQ = 200
H4 = 4 * HIDDEN_DIM
CHUNK = 4
NCHUNK = SEQ // CHUNK


def _lstm_kernel(urls_ref, urlsr_ref, emb_ref, wihf_ref, whhf_ref, bf_ref,
                 wihb_ref, whhb_ref, bb_ref,
                 fcw_ref, fcb_ref, fc1w_ref, fc1b_ref,
                 fc2w_ref, fc2b_ref, fc3w_ref, fc3b_ref,
                 out_ref, aux_ref,
                 mf_scr, mb_scr, gifa_scr, giba_scr, gifb_scr, gibb_scr):
    f32 = jnp.float32
    bf16 = jnp.bfloat16
    rows = CHUNK * BATCH

    emb = emb_ref[...]
    mf_scr[...] = (jnp.dot(emb, wihf_ref[...], preferred_element_type=f32)
                   + bf_ref[...]).astype(bf16)
    mb_scr[...] = (jnp.dot(emb, wihb_ref[...], preferred_element_type=f32)
                   + bb_ref[...]).astype(bf16)

    iota = jax.lax.broadcasted_iota(jnp.int32, (rows, INPUT_DIM), 1)

    whhf = whhf_ref[...]
    whhb = whhb_ref[...]
    mf = mf_scr[...]
    mb = mb_scr[...]

    def sig(x):
        return 0.5 * jnp.tanh(0.5 * x) + 0.5

    def step_dir(gin, h, c, whh):
        gates = jnp.dot(h, whh, preferred_element_type=f32) + gin.astype(f32)
        i = sig(gates[:, 0:HIDDEN_DIM])
        f = sig(gates[:, HIDDEN_DIM:2 * HIDDEN_DIM])
        g = jnp.tanh(gates[:, 2 * HIDDEN_DIM:3 * HIDDEN_DIM])
        o = sig(gates[:, 3 * HIDDEN_DIM:])
        c = f * c + i * g
        h = (o * jnp.tanh(c)).astype(bf16)
        return h, c

    def gin_chunk(k, dst_f, dst_b):
        ids_f = urls_ref[pl.ds(k * rows, rows), :]
        ids_b = urlsr_ref[pl.ds(k * rows, rows), :]
        dst_f[...] = jnp.dot((ids_f == iota).astype(bf16), mf,
                             preferred_element_type=f32).astype(bf16)
        dst_b[...] = jnp.dot((ids_b == iota).astype(bf16), mb,
                             preferred_element_type=f32).astype(bf16)

    def steps(gf_scr, gb_scr, carry):
        hf, cf, hb, cb = carry
        for j in range(CHUNK):
            gf = gf_scr[j * BATCH:(j + 1) * BATCH, :]
            gb = gb_scr[j * BATCH:(j + 1) * BATCH, :]
            hf, cf = step_dir(gf, hf, cf, whhf)
            hb, cb = step_dir(gb, hb, cb, whhb)
        return hf, cf, hb, cb

    def trip(m, carry):
        gin_chunk(2 * m + 1, gifb_scr, gibb_scr)
        carry = steps(gifa_scr, giba_scr, carry)
        gin_chunk(jnp.minimum(2 * m + 2, NCHUNK - 1), gifa_scr, giba_scr)
        carry = steps(gifb_scr, gibb_scr, carry)
        return carry

    gin_chunk(0, gifa_scr, giba_scr)
    hf = jnp.zeros((BATCH, HIDDEN_DIM), bf16)
    hb = jnp.zeros((BATCH, HIDDEN_DIM), bf16)
    cf = jnp.zeros((BATCH, HIDDEN_DIM), f32)
    cb = jnp.zeros((BATCH, HIDDEN_DIM), f32)
    hf, cf, hb, cb = jax.lax.fori_loop(0, NCHUNK // 2, trip,
                                       (hf, cf, hb, cb))

    hidden = jnp.concatenate([hf, hb], axis=1)
    aux_ref[...] = jnp.dot(hidden, fcw_ref[...], preferred_element_type=f32) + fcb_ref[...]
    o1 = (jnp.dot(hidden, fc1w_ref[...], preferred_element_type=f32)
          + fc1b_ref[...]).astype(bf16)
    o2 = (jnp.dot(o1, fc2w_ref[...], preferred_element_type=f32)
          + fc2b_ref[...]).astype(bf16)
    out_ref[...] = jnp.dot(o2, fc3w_ref[...], preferred_element_type=f32) + fc3b_ref[...]


@functools.partial(jax.jit, static_argnames=("interpret",))
def _run(urls, emb_table, W_ih_f, W_hh_f, b_f, W_ih_b, W_hh_b, b_b,
         fc_w, fc_b, fc1_w, fc1_b, fc2_w, fc2_b, fc3_w, fc3_b,
         interpret=False):
    urls_flat = urls.T.reshape(SEQ * BATCH, 1).astype(jnp.int32)
    urls_rev = urls.T[::-1].reshape(SEQ * BATCH, 1).astype(jnp.int32)
    f32 = jnp.float32
    bf16 = jnp.bfloat16
    args = (
        urls_flat,
        urls_rev,
        emb_table,
        W_ih_f.T.astype(bf16), W_hh_f.T.astype(bf16), b_f.reshape(1, H4),
        W_ih_b.T.astype(bf16), W_hh_b.T.astype(bf16), b_b.reshape(1, H4),
        fc_w.T.astype(bf16), fc_b.reshape(1, 1),
        fc1_w.T.astype(bf16), fc1_b.reshape(1, H4),
        fc2_w.T.astype(bf16), fc2_b.reshape(1, 2 * HIDDEN_DIM),
        fc3_w.T.astype(bf16), fc3_b.reshape(1, 2),
    )
    out, aux = pl.pallas_call(
        _lstm_kernel,
        out_shape=(
            jax.ShapeDtypeStruct((BATCH, 2), f32),
            jax.ShapeDtypeStruct((BATCH, 1), f32),
        ),
        scratch_shapes=[
            pltpu.VMEM((INPUT_DIM, H4), bf16),
            pltpu.VMEM((INPUT_DIM, H4), bf16),
            pltpu.VMEM((CHUNK * BATCH, H4), bf16),
            pltpu.VMEM((CHUNK * BATCH, H4), bf16),
            pltpu.VMEM((CHUNK * BATCH, H4), bf16),
            pltpu.VMEM((CHUNK * BATCH, H4), bf16),
        ],
        interpret=interpret,
    )(*args)
    return out, aux[:, 0]


def kernel(urls, emb_table, W_ih_f, W_hh_f, b_f, W_ih_b, W_hh_b, b_b,
           fc_w, fc_b, fc1_w, fc1_b, fc2_w, fc2_b, fc3_w, fc3_b):
    return _run(urls, emb_table, W_ih_f, W_hh_f, b_f, W_ih_b, W_hh_b, b_b,
                fc_w, fc_b, fc1_w, fc1_b, fc2_w, fc2_b, fc3_w, fc3_b)


# speedup vs baseline: 1.0717x; 1.0687x over previous
import functools

import jax
import jax.numpy as jnp
from jax.experimental import pallas as pl
from jax.experimental.pallas import tpu as pltpu

INPUT_DIM = 256
EMB_DIM = 128
HIDDEN_DIM = 512
BATCH = 128
SEQ = 200
H4 = 4 * HIDDEN_DIM
CHUNK = 8


def _lstm_kernel(urls_ref, emb_ref, wihf_ref, whhf_ref, bf_ref,
                 wihb_ref, whhb_ref, bb_ref,
                 fcw_ref, fcb_ref, fc1w_ref, fc1b_ref,
                 fc2w_ref, fc2b_ref, fc3w_ref, fc3b_ref,
                 out_ref, aux_ref,
                 mf_scr, mb_scr, gif_scr, gib_scr):
    f32 = jnp.float32
    bf16 = jnp.bfloat16
    rows = CHUNK * BATCH

    emb = emb_ref[...]
    mf_scr[...] = (jnp.dot(emb, wihf_ref[...], preferred_element_type=f32)
                   + bf_ref[...]).astype(bf16)
    mb_scr[...] = (jnp.dot(emb, wihb_ref[...], preferred_element_type=f32)
                   + bb_ref[...]).astype(bf16)

    iota = jax.lax.broadcasted_iota(jnp.int32, (rows, INPUT_DIM), 1)

    whhf = whhf_ref[...]
    whhb = whhb_ref[...]
    mf = mf_scr[...]
    mb = mb_scr[...]

    def sig(x):
        return 0.5 * jnp.tanh(0.5 * x) + 0.5

    def step_dir(gin, h, c, whh):
        gates = jnp.dot(h, whh, preferred_element_type=f32) + gin.astype(f32)
        i = sig(gates[:, 0:HIDDEN_DIM])
        f = sig(gates[:, HIDDEN_DIM:2 * HIDDEN_DIM])
        g = jnp.tanh(gates[:, 2 * HIDDEN_DIM:3 * HIDDEN_DIM])
        o = sig(gates[:, 3 * HIDDEN_DIM:])
        c = f * c + i * g
        h = (o * jnp.tanh(c)).astype(bf16)
        return h, c

    def chunk_body(k, carry):
        ids_f = urls_ref[pl.ds(k * rows, rows), :]
        ids_b = urls_ref[pl.ds((SEQ * BATCH) - (k + 1) * rows, rows), :]
        gif_scr[...] = jnp.dot((ids_f == iota).astype(bf16), mf,
                               preferred_element_type=f32).astype(bf16)
        gib_scr[...] = jnp.dot((ids_b == iota).astype(bf16), mb,
                               preferred_element_type=f32).astype(bf16)

        hf, cf, hb, cb = carry
        for j in range(CHUNK):
            gf = gif_scr[j * BATCH:(j + 1) * BATCH, :]
            gb = gib_scr[(CHUNK - 1 - j) * BATCH:(CHUNK - j) * BATCH, :]
            hf, cf = step_dir(gf, hf, cf, whhf)
            hb, cb = step_dir(gb, hb, cb, whhb)
        return hf, cf, hb, cb

    hf = jnp.zeros((BATCH, HIDDEN_DIM), bf16)
    hb = jnp.zeros((BATCH, HIDDEN_DIM), bf16)
    cf = jnp.zeros((BATCH, HIDDEN_DIM), f32)
    cb = jnp.zeros((BATCH, HIDDEN_DIM), f32)
    hf, cf, hb, cb = jax.lax.fori_loop(0, SEQ // CHUNK, chunk_body,
                                       (hf, cf, hb, cb))

    hidden = jnp.concatenate([hf, hb], axis=1)
    aux_ref[...] = jnp.dot(hidden, fcw_ref[...], preferred_element_type=f32) + fcb_ref[...]
    o1 = (jnp.dot(hidden, fc1w_ref[...], preferred_element_type=f32)
          + fc1b_ref[...]).astype(bf16)
    o2 = (jnp.dot(o1, fc2w_ref[...], preferred_element_type=f32)
          + fc2b_ref[...]).astype(bf16)
    out_ref[...] = jnp.dot(o2, fc3w_ref[...], preferred_element_type=f32) + fc3b_ref[...]


@functools.partial(jax.jit, static_argnames=("interpret",))
def _run(urls, emb_table, W_ih_f, W_hh_f, b_f, W_ih_b, W_hh_b, b_b,
         fc_w, fc_b, fc1_w, fc1_b, fc2_w, fc2_b, fc3_w, fc3_b,
         interpret=False):
    urls_flat = urls.T.reshape(SEQ * BATCH, 1).astype(jnp.int32)
    f32 = jnp.float32
    bf16 = jnp.bfloat16
    args = (
        urls_flat,
        emb_table,
        W_ih_f.T, W_hh_f.T.astype(bf16), b_f.reshape(1, H4),
        W_ih_b.T, W_hh_b.T.astype(bf16), b_b.reshape(1, H4),
        fc_w.T.astype(bf16), fc_b.reshape(1, 1),
        fc1_w.T.astype(bf16), fc1_b.reshape(1, H4),
        fc2_w.T.astype(bf16), fc2_b.reshape(1, 2 * HIDDEN_DIM),
        fc3_w.T.astype(bf16), fc3_b.reshape(1, 2),
    )
    out, aux = pl.pallas_call(
        _lstm_kernel,
        out_shape=(
            jax.ShapeDtypeStruct((BATCH, 2), f32),
            jax.ShapeDtypeStruct((BATCH, 1), f32),
        ),
        scratch_shapes=[
            pltpu.VMEM((INPUT_DIM, H4), bf16),
            pltpu.VMEM((INPUT_DIM, H4), bf16),
            pltpu.VMEM((CHUNK * BATCH, H4), bf16),
            pltpu.VMEM((CHUNK * BATCH, H4), bf16),
        ],
        interpret=interpret,
    )(*args)
    return out, aux[:, 0]


def kernel(urls, emb_table, W_ih_f, W_hh_f, b_f, W_ih_b, W_hh_b, b_b,
           fc_w, fc_b, fc1_w, fc1_b, fc2_w, fc2_b, fc3_w, fc3_b):
    return _run(urls, emb_table, W_ih_f, W_hh_f, b_f, W_ih_b, W_hh_b, b_b,
                fc_w, fc_b, fc1_w, fc1_b, fc2_w, fc2_b, fc3_w, fc3_b)
